# baseline (speedup 1.0000x reference)
import functools

import jax
import jax.numpy as jnp
from jax import lax
from jax.experimental import pallas as pl
from jax.experimental.pallas import tpu as pltpu
from jax.experimental.pallas import tpu_sc as plsc

G = 64
NS = 16
NC = 2
K = 128
GN = 10
BN_ROWS = 2000


def _sc_agg(table, comb, zeros, *, n, edge_split):
    ch = table.shape[1]
    ncht = comb.shape[0] if edge_split else comb.shape[0] // 2
    nch_w = ncht // (NS * NC) if edge_split else ncht // NS
    npair = nch_w // 2
    zr = (n // NS) & ~7
    tail = n - NS * zr
    mesh = plsc.VectorSubcoreMesh(core_axis_name="c", subcore_axis_name="s")

    @functools.partial(
        pl.kernel,
        out_type=jax.ShapeDtypeStruct((2 * n, ch), jnp.float32),
        mesh=mesh,
        scratch_types=[
            [pltpu.VMEM((GN, 2, K), jnp.int32)] * 4,
            [pltpu.VMEM((K, ch), jnp.float32)] * 2,
            pltpu.VMEM_SHARED((n + 16, ch), jnp.float32),
            [pltpu.SemaphoreType.DMA] * 4,
            [pltpu.SemaphoreType.DMA] * 2,
        ],
    )
    def body(comb_ref, t_ref, z_ref, out_ref, ibuf, rows, aggsh, isem, gsem):
        c = lax.axis_index("c")
        s = lax.axis_index("s")
        pltpu.sync_copy(z_ref.at[pl.ds(0, zr)],
                        aggsh.at[pl.ds(pl.multiple_of(s * zr, 8), zr)])
        @pl.when(s == 0)
        def _():
            pltpu.sync_copy(z_ref.at[pl.ds(0, tail)],
                            aggsh.at[pl.ds(NS * zr, tail)])
        plsc.subcore_barrier()

        if edge_split:
            base = (s * NC + c) * nch_w
        else:
            base = c * ncht + s * nch_w

        ng = nch_w // GN

        def fire_ig(b, grp_i):
            pltpu.async_copy(comb_ref.at[pl.ds(base + grp_i * GN, GN)],
                             ibuf[b], isem[b])

        def wait_ig(b):
            pltpu.make_async_copy(comb_ref.at[pl.ds(0, GN)], ibuf[b],
                                  isem[b]).wait()

        def fire_gather(p, idx_row):
            pltpu.async_copy(t_ref.at[idx_row.at[0]], rows[p], gsem[p])

        def wait_gather(p):
            pltpu.make_async_copy(t_ref.at[pl.ds(0, K)], rows[p], gsem[p]).wait()

        def process_group(cur, nxt, gbase):
            for t in range(GN):
                p = t % 2
                wait_gather(p)
                if t < GN - 1:
                    fire_gather(1 - p, cur.at[t + 1])
                else:
                    @pl.when(gbase + GN < nch_w)
                    def _():
                        fire_gather(1 - p, nxt.at[0])
                pltpu.sync_copy(rows[p], aggsh.at[cur.at[t].at[1]], add=True)

        fire_ig(0, 0)
        fire_ig(1, 1)
        wait_ig(0)
        fire_gather(0, ibuf[0].at[0])

        def gquad(v, carry):
            for w in range(4):
                g = 4 * v + w
                @pl.when(g + 2 < ng)
                def _():
                    fire_ig((w + 2) % 4, g + 2)
                @pl.when(g + 1 < ng)
                def _():
                    wait_ig((w + 1) % 4)
                process_group(ibuf[w], ibuf[(w + 1) % 4], g * GN)
            return carry

        lax.fori_loop(0, ng // 4, gquad, 0)
        plsc.subcore_barrier()
        pltpu.sync_copy(aggsh.at[pl.ds(pl.multiple_of(s * zr, 8), zr)],
                        out_ref.at[pl.ds(pl.multiple_of(c * n + s * zr, 8), zr)])
        @pl.when(s == 0)
        def _():
            pltpu.sync_copy(aggsh.at[pl.ds(NS * zr, tail)],
                            out_ref.at[pl.ds(pl.multiple_of(c * n + NS * zr, 8), tail)])

    return body(comb, table, zeros)


def _pass_a0(x, p01, epsp, w1, b1, *, n):
    cin = x.shape[1]
    h2 = w1.shape[1]
    nb = n // BN_ROWS

    def body(eps_ref, x_ref, a0_ref, a1_ref, w_ref, b_ref, z_ref, s_ref, q_ref):
        i = pl.program_id(0)
        u = x_ref[...] * eps_ref[0, 0] + a0_ref[...] + a1_ref[...]
        z = jnp.dot(u, w_ref[...], preferred_element_type=jnp.float32) + b_ref[...]
        z_ref[...] = z
        @pl.when(i == 0)
        def _():
            s_ref[...] = jnp.zeros_like(s_ref)
            q_ref[...] = jnp.zeros_like(q_ref)
        s_ref[...] += jnp.sum(z, 0, keepdims=True)
        q_ref[...] += jnp.sum(z * z, 0, keepdims=True)

    return pl.pallas_call(
        body,
        grid=(nb,),
        in_specs=[
            pl.BlockSpec((1, 1), lambda i: (0, 0), memory_space=pltpu.SMEM),
            pl.BlockSpec((BN_ROWS, cin), lambda i: (i, 0)),
            pl.BlockSpec((BN_ROWS, cin), lambda i: (i, 0)),
            pl.BlockSpec((BN_ROWS, cin), lambda i: (nb + i, 0)),
            pl.BlockSpec((cin, h2), lambda i: (0, 0)),
            pl.BlockSpec((1, h2), lambda i: (0, 0)),
        ],
        out_specs=[
            pl.BlockSpec((BN_ROWS, h2), lambda i: (i, 0)),
            pl.BlockSpec((1, h2), lambda i: (0, 0)),
            pl.BlockSpec((1, h2), lambda i: (0, 0)),
        ],
        out_shape=[
            jax.ShapeDtypeStruct((n, h2), jnp.float32),
            jax.ShapeDtypeStruct((1, h2), jnp.float32),
            jax.ShapeDtypeStruct((1, h2), jnp.float32),
        ],
    )(epsp, x, p01, p01, w1, b1)


def _pass_a(hsplit, agg, epsp, w1, b1, *, n):
    ch = hsplit.shape[1]
    h2 = w1.shape[1]
    nb = n // BN_ROWS

    def body(eps_ref, h0, h1, a0, a1, w_ref, b_ref, z_ref, s_ref, q_ref):
        i = pl.program_id(0)
        ep = eps_ref[0, 0]
        u0 = h0[...] * ep + a0[...]
        u1 = h1[...] * ep + a1[...]
        z = (jnp.dot(u0, w_ref[:ch], preferred_element_type=jnp.float32)
             + jnp.dot(u1, w_ref[ch:], preferred_element_type=jnp.float32)
             + b_ref[...])
        z_ref[...] = z
        @pl.when(i == 0)
        def _():
            s_ref[...] = jnp.zeros_like(s_ref)
            q_ref[...] = jnp.zeros_like(q_ref)
        s_ref[...] += jnp.sum(z, 0, keepdims=True)
        q_ref[...] += jnp.sum(z * z, 0, keepdims=True)

    return pl.pallas_call(
        body,
        grid=(nb,),
        in_specs=[
            pl.BlockSpec((1, 1), lambda i: (0, 0), memory_space=pltpu.SMEM),
            pl.BlockSpec((BN_ROWS, ch), lambda i: (i, 0)),
            pl.BlockSpec((BN_ROWS, ch), lambda i: (nb + i, 0)),
            pl.BlockSpec((BN_ROWS, ch), lambda i: (i, 0)),
            pl.BlockSpec((BN_ROWS, ch), lambda i: (nb + i, 0)),
            pl.BlockSpec((2 * ch, h2), lambda i: (0, 0)),
            pl.BlockSpec((1, h2), lambda i: (0, 0)),
        ],
        out_specs=[
            pl.BlockSpec((BN_ROWS, h2), lambda i: (i, 0)),
            pl.BlockSpec((1, h2), lambda i: (0, 0)),
            pl.BlockSpec((1, h2), lambda i: (0, 0)),
        ],
        out_shape=[
            jax.ShapeDtypeStruct((n, h2), jnp.float32),
            jax.ShapeDtypeStruct((1, h2), jnp.float32),
            jax.ShapeDtypeStruct((1, h2), jnp.float32),
        ],
    )(epsp, hsplit, hsplit, agg, agg, w1, b1)


def _pass_b(z1, s1, q1, g1, bt1, w2, b2, *, n):
    h2 = z1.shape[1]
    ho = w2.shape[1]
    nb = n // BN_ROWS

    def body(z1_ref, s_ref, q_ref, g_ref, bt_ref, w_ref, b_ref,
             z2_ref, s2_ref, q2_ref):
        i = pl.program_id(0)
        m = s_ref[...] / n
        v = q_ref[...] / n - m * m
        sc = g_ref[...] * lax.rsqrt(v + 1e-5)
        sh = bt_ref[...] - m * sc
        y = jnp.maximum(z1_ref[...] * sc + sh, 0.0)
        z2 = jnp.dot(y, w_ref[...], preferred_element_type=jnp.float32) + b_ref[...]
        z2_ref[...] = z2
        @pl.when(i == 0)
        def _():
            s2_ref[...] = jnp.zeros_like(s2_ref)
            q2_ref[...] = jnp.zeros_like(q2_ref)
        s2_ref[...] += jnp.sum(z2, 0, keepdims=True)
        q2_ref[...] += jnp.sum(z2 * z2, 0, keepdims=True)

    return pl.pallas_call(
        body,
        grid=(nb,),
        in_specs=[
            pl.BlockSpec((BN_ROWS, h2), lambda i: (i, 0)),
            pl.BlockSpec((1, h2), lambda i: (0, 0)),
            pl.BlockSpec((1, h2), lambda i: (0, 0)),
            pl.BlockSpec((1, h2), lambda i: (0, 0)),
            pl.BlockSpec((1, h2), lambda i: (0, 0)),
            pl.BlockSpec((h2, ho), lambda i: (0, 0)),
            pl.BlockSpec((1, ho), lambda i: (0, 0)),
        ],
        out_specs=[
            pl.BlockSpec((BN_ROWS, ho), lambda i: (i, 0)),
            pl.BlockSpec((1, ho), lambda i: (0, 0)),
            pl.BlockSpec((1, ho), lambda i: (0, 0)),
        ],
        out_shape=[
            jax.ShapeDtypeStruct((n, ho), jnp.float32),
            jax.ShapeDtypeStruct((1, ho), jnp.float32),
            jax.ShapeDtypeStruct((1, ho), jnp.float32),
        ],
    )(z1, s1, q1, g1, bt1, w2, b2)


def _pass_c_split(z2, s2, q2, g, bt, *, n):
    h = z2.shape[1]
    ch = h // 2
    nb = n // BN_ROWS

    def body(z_ref, s_ref, q_ref, g_ref, bt_ref, out_ref):
        m = s_ref[...] / n
        v = q_ref[...] / n - m * m
        sc = g_ref[...] * lax.rsqrt(v + 1e-5)
        sh = bt_ref[...] - m * sc
        out_ref[...] = jnp.maximum(z_ref[...] * sc + sh, 0.0)

    return pl.pallas_call(
        body,
        grid=(nb, 2),
        in_specs=[
            pl.BlockSpec((BN_ROWS, ch), lambda i, hh: (i, hh)),
            pl.BlockSpec((1, ch), lambda i, hh: (0, hh)),
            pl.BlockSpec((1, ch), lambda i, hh: (0, hh)),
            pl.BlockSpec((1, ch), lambda i, hh: (0, hh)),
            pl.BlockSpec((1, ch), lambda i, hh: (0, hh)),
        ],
        out_specs=pl.BlockSpec((BN_ROWS, ch), lambda i, hh: (hh * nb + i, 0)),
        out_shape=jax.ShapeDtypeStruct((2 * n, ch), jnp.float32),
    )(z2, s2, q2, g, bt)


def _pass_c_pool(z2, s2, q2, g, bt, batch2d, *, n):
    h = z2.shape[1]
    nb = n // BN_ROWS

    def body(z_ref, s_ref, q_ref, g_ref, bt_ref, b_ref, out_ref):
        i = pl.program_id(0)
        m = s_ref[...] / n
        v = q_ref[...] / n - m * m
        sc = g_ref[...] * lax.rsqrt(v + 1e-5)
        sh = bt_ref[...] - m * sc
        hrows = jnp.maximum(z_ref[...] * sc + sh, 0.0)
        seg = lax.broadcasted_iota(jnp.int32, (G, BN_ROWS), 0)
        onehot = (seg == b_ref[0]).astype(jnp.float32)
        @pl.when(i == 0)
        def _():
            out_ref[...] = jnp.zeros_like(out_ref)
        out_ref[...] += jnp.dot(onehot, hrows, preferred_element_type=jnp.float32)

    return pl.pallas_call(
        body,
        grid=(nb,),
        in_specs=[
            pl.BlockSpec((BN_ROWS, h), lambda i: (i, 0)),
            pl.BlockSpec((1, h), lambda i: (0, 0)),
            pl.BlockSpec((1, h), lambda i: (0, 0)),
            pl.BlockSpec((1, h), lambda i: (0, 0)),
            pl.BlockSpec((1, h), lambda i: (0, 0)),
            pl.BlockSpec((1, 1, BN_ROWS), lambda i: (i, 0, 0)),
        ],
        out_specs=pl.BlockSpec((G, h), lambda i: (0, 0)),
        out_shape=jax.ShapeDtypeStruct((G, h), jnp.float32),
    )(z2, s2, q2, g, bt, batch2d)


def _head(pooled, w1, b1, g1, bt1, w2, b2):
    h = pooled.shape[1]
    out = w2.shape[1]

    def body(p_ref, w1_ref, b1_ref, g_ref, bt_ref, w2_ref, b2_ref, o_ref):
        y = jnp.dot(p_ref[...], w1_ref[...],
                    preferred_element_type=jnp.float32) + b1_ref[...]
        m = jnp.mean(y, 0, keepdims=True)
        v = jnp.mean(y * y, 0, keepdims=True) - m * m
        y = jnp.maximum(g_ref[...] * (y - m) * lax.rsqrt(v + 1e-5) + bt_ref[...], 0.0)
        y2 = jnp.dot(y, w2_ref[...], preferred_element_type=jnp.float32) + b2_ref[...]
        mx = jnp.max(y2, axis=-1, keepdims=True)
        lse = mx + jnp.log(jnp.sum(jnp.exp(y2 - mx), axis=-1, keepdims=True))
        o_ref[...] = y2 - lse

    return pl.pallas_call(
        body,
        out_shape=jax.ShapeDtypeStruct((G, out), jnp.float32),
    )(pooled, w1, b1, g1, bt1, w2, b2)


def kernel(x, edge_index, batch, params):
    n = x.shape[0]
    e = edge_index.shape[1]
    src = edge_index[0]
    dst = edge_index[1]
    grp = 4 * GN * K * NS * NC
    e2 = -(-e // grp) * grp
    src_p = jnp.concatenate([src, jnp.arange(e2 - e, dtype=jnp.int32) % n])
    dst_p = jnp.concatenate([dst, n + (jnp.arange(e2 - e, dtype=jnp.int32) % 16)])
    sb = src_p.reshape(-1, 1, K)
    db = dst_p.reshape(-1, 1, K)
    comb0 = jnp.concatenate([sb, db], 1)
    comb = jnp.concatenate([comb0,
                            jnp.concatenate([sb + n, db], 1)], 0)
    zeros = jnp.zeros((n // NS, 128), jnp.float32)
    batch3d = batch.reshape(n // BN_ROWS, 1, BN_ROWS)

    def row(a):
        return a.reshape(1, -1)

    hsplit = None
    z2 = s2 = q2 = None
    for i in range(3):
        p = params[f"conv{i}"]
        epsp = (1.0 + p["eps"]).reshape(1, 1)
        if i == 0:
            p01 = _sc_agg(x, comb0, zeros, n=n, edge_split=True)
            z1, s1, q1 = _pass_a0(x, p01, epsp, p["W1"], row(p["b1"]), n=n)
        else:
            hsplit = _pass_c_split(z2, s2, q2, row(params[f"bn{i-1}_g"]),
                                   row(params[f"bn{i-1}_b"]), n=n)
            agg = _sc_agg(hsplit, comb, zeros, n=n, edge_split=False)
            z1, s1, q1 = _pass_a(hsplit, agg, epsp, p["W1"], row(p["b1"]), n=n)
        z2, s2, q2 = _pass_b(z1, s1, q1, row(p["g1"]), row(p["bt1"]),
                             p["W2"], row(p["b2"]), n=n)

    pooled = _pass_c_pool(z2, s2, q2, row(params["bn2_g"]), row(params["bn2_b"]),
                          batch3d, n=n)
    return _head(pooled, params["lin1_W"], row(params["lin1_b"]),
                 row(params["bn1_g"]), row(params["bn1_b"]),
                 params["lin2_W"], row(params["lin2_b"]))


# speedup vs baseline: 1.0011x; 1.0011x over previous
import functools

import jax
import jax.numpy as jnp
from jax import lax
from jax.experimental import pallas as pl
from jax.experimental.pallas import tpu as pltpu
from jax.experimental.pallas import tpu_sc as plsc

G = 64
NS = 16
NC = 2
K = 128
GN = 10
BN_ROWS = 2000


def _sc_agg(table, comb, zeros, *, n, edge_split):
    ch = table.shape[1]
    ncht = comb.shape[0] if edge_split else comb.shape[0] // 2
    nch_w = ncht // (NS * NC) if edge_split else ncht // NS
    npair = nch_w // 2
    zr = (n // NS) & ~7
    tail = n - NS * zr
    mesh = plsc.VectorSubcoreMesh(core_axis_name="c", subcore_axis_name="s")

    @functools.partial(
        pl.kernel,
        out_type=jax.ShapeDtypeStruct((2 * n, ch), jnp.float32),
        mesh=mesh,
        scratch_types=[
            [pltpu.VMEM((GN, 2, K), jnp.int32)] * 4,
            [pltpu.VMEM((K, ch), jnp.float32)] * 2,
            pltpu.VMEM_SHARED((n + 16, ch), jnp.float32),
            [pltpu.SemaphoreType.DMA] * 4,
            [pltpu.SemaphoreType.DMA] * 2,
            [pltpu.SemaphoreType.DMA] * 2,
        ],
    )
    def body(comb_ref, t_ref, z_ref, out_ref, ibuf, rows, aggsh, isem, gsem, ssem):
        c = lax.axis_index("c")
        s = lax.axis_index("s")
        pltpu.sync_copy(z_ref.at[pl.ds(0, zr)],
                        aggsh.at[pl.ds(pl.multiple_of(s * zr, 8), zr)])
        @pl.when(s == 0)
        def _():
            pltpu.sync_copy(z_ref.at[pl.ds(0, tail)],
                            aggsh.at[pl.ds(NS * zr, tail)])
        plsc.subcore_barrier()

        if edge_split:
            base = (s * NC + c) * nch_w
        else:
            base = c * ncht + s * nch_w

        ng = nch_w // GN

        def fire_ig(b, grp_i):
            pltpu.async_copy(comb_ref.at[pl.ds(base + grp_i * GN, GN)],
                             ibuf[b], isem[b])

        def wait_ig(b):
            pltpu.make_async_copy(comb_ref.at[pl.ds(0, GN)], ibuf[b],
                                  isem[b]).wait()

        def fire_gather(p, idx_row):
            pltpu.async_copy(t_ref.at[idx_row.at[0]], rows[p], gsem[p])

        def wait_gather(p):
            pltpu.make_async_copy(t_ref.at[pl.ds(0, K)], rows[p], gsem[p]).wait()

        def wait_scat(p):
            pltpu.make_async_copy(rows[p], aggsh.at[pl.ds(0, K)], ssem[p]).wait()

        def process_group(cur, nxt, gbase):
            for t in range(GN):
                p = t % 2
                wait_gather(p)
                @pl.when(gbase + t >= 1)
                def _():
                    wait_scat(1 - p)
                if t < GN - 1:
                    fire_gather(1 - p, cur.at[t + 1])
                else:
                    @pl.when(gbase + GN < nch_w)
                    def _():
                        fire_gather(1 - p, nxt.at[0])
                pltpu.async_copy(rows[p], aggsh.at[cur.at[t].at[1]], ssem[p],
                                 add=True)

        fire_ig(0, 0)
        fire_ig(1, 1)
        wait_ig(0)
        fire_gather(0, ibuf[0].at[0])

        def gquad(v, carry):
            for w in range(4):
                g = 4 * v + w
                @pl.when(g + 2 < ng)
                def _():
                    fire_ig((w + 2) % 4, g + 2)
                @pl.when(g + 1 < ng)
                def _():
                    wait_ig((w + 1) % 4)
                process_group(ibuf[w], ibuf[(w + 1) % 4], g * GN)
            return carry

        lax.fori_loop(0, ng // 4, gquad, 0)
        wait_scat((nch_w - 1) % 2)
        plsc.subcore_barrier()
        pltpu.sync_copy(aggsh.at[pl.ds(pl.multiple_of(s * zr, 8), zr)],
                        out_ref.at[pl.ds(pl.multiple_of(c * n + s * zr, 8), zr)])
        @pl.when(s == 0)
        def _():
            pltpu.sync_copy(aggsh.at[pl.ds(NS * zr, tail)],
                            out_ref.at[pl.ds(pl.multiple_of(c * n + NS * zr, 8), tail)])

    return body(comb, table, zeros)


def _pass_a0(x, p01, epsp, w1, b1, *, n):
    cin = x.shape[1]
    h2 = w1.shape[1]
    nb = n // BN_ROWS

    def body(eps_ref, x_ref, a0_ref, a1_ref, w_ref, b_ref, z_ref, s_ref, q_ref):
        i = pl.program_id(0)
        u = x_ref[...] * eps_ref[0, 0] + a0_ref[...] + a1_ref[...]
        z = jnp.dot(u, w_ref[...], preferred_element_type=jnp.float32) + b_ref[...]
        z_ref[...] = z
        @pl.when(i == 0)
        def _():
            s_ref[...] = jnp.zeros_like(s_ref)
            q_ref[...] = jnp.zeros_like(q_ref)
        s_ref[...] += jnp.sum(z, 0, keepdims=True)
        q_ref[...] += jnp.sum(z * z, 0, keepdims=True)

    return pl.pallas_call(
        body,
        grid=(nb,),
        in_specs=[
            pl.BlockSpec((1, 1), lambda i: (0, 0), memory_space=pltpu.SMEM),
            pl.BlockSpec((BN_ROWS, cin), lambda i: (i, 0)),
            pl.BlockSpec((BN_ROWS, cin), lambda i: (i, 0)),
            pl.BlockSpec((BN_ROWS, cin), lambda i: (nb + i, 0)),
            pl.BlockSpec((cin, h2), lambda i: (0, 0)),
            pl.BlockSpec((1, h2), lambda i: (0, 0)),
        ],
        out_specs=[
            pl.BlockSpec((BN_ROWS, h2), lambda i: (i, 0)),
            pl.BlockSpec((1, h2), lambda i: (0, 0)),
            pl.BlockSpec((1, h2), lambda i: (0, 0)),
        ],
        out_shape=[
            jax.ShapeDtypeStruct((n, h2), jnp.float32),
            jax.ShapeDtypeStruct((1, h2), jnp.float32),
            jax.ShapeDtypeStruct((1, h2), jnp.float32),
        ],
    )(epsp, x, p01, p01, w1, b1)


def _pass_a(hsplit, agg, epsp, w1, b1, *, n):
    ch = hsplit.shape[1]
    h2 = w1.shape[1]
    nb = n // BN_ROWS

    def body(eps_ref, h0, h1, a0, a1, w_ref, b_ref, z_ref, s_ref, q_ref):
        i = pl.program_id(0)
        ep = eps_ref[0, 0]
        u0 = h0[...] * ep + a0[...]
        u1 = h1[...] * ep + a1[...]
        z = (jnp.dot(u0, w_ref[:ch], preferred_element_type=jnp.float32)
             + jnp.dot(u1, w_ref[ch:], preferred_element_type=jnp.float32)
             + b_ref[...])
        z_ref[...] = z
        @pl.when(i == 0)
        def _():
            s_ref[...] = jnp.zeros_like(s_ref)
            q_ref[...] = jnp.zeros_like(q_ref)
        s_ref[...] += jnp.sum(z, 0, keepdims=True)
        q_ref[...] += jnp.sum(z * z, 0, keepdims=True)

    return pl.pallas_call(
        body,
        grid=(nb,),
        in_specs=[
            pl.BlockSpec((1, 1), lambda i: (0, 0), memory_space=pltpu.SMEM),
            pl.BlockSpec((BN_ROWS, ch), lambda i: (i, 0)),
            pl.BlockSpec((BN_ROWS, ch), lambda i: (nb + i, 0)),
            pl.BlockSpec((BN_ROWS, ch), lambda i: (i, 0)),
            pl.BlockSpec((BN_ROWS, ch), lambda i: (nb + i, 0)),
            pl.BlockSpec((2 * ch, h2), lambda i: (0, 0)),
            pl.BlockSpec((1, h2), lambda i: (0, 0)),
        ],
        out_specs=[
            pl.BlockSpec((BN_ROWS, h2), lambda i: (i, 0)),
            pl.BlockSpec((1, h2), lambda i: (0, 0)),
            pl.BlockSpec((1, h2), lambda i: (0, 0)),
        ],
        out_shape=[
            jax.ShapeDtypeStruct((n, h2), jnp.float32),
            jax.ShapeDtypeStruct((1, h2), jnp.float32),
            jax.ShapeDtypeStruct((1, h2), jnp.float32),
        ],
    )(epsp, hsplit, hsplit, agg, agg, w1, b1)


def _pass_b(z1, s1, q1, g1, bt1, w2, b2, *, n):
    h2 = z1.shape[1]
    ho = w2.shape[1]
    nb = n // BN_ROWS

    def body(z1_ref, s_ref, q_ref, g_ref, bt_ref, w_ref, b_ref,
             z2_ref, s2_ref, q2_ref):
        i = pl.program_id(0)
        m = s_ref[...] / n
        v = q_ref[...] / n - m * m
        sc = g_ref[...] * lax.rsqrt(v + 1e-5)
        sh = bt_ref[...] - m * sc
        y = jnp.maximum(z1_ref[...] * sc + sh, 0.0)
        z2 = jnp.dot(y, w_ref[...], preferred_element_type=jnp.float32) + b_ref[...]
        z2_ref[...] = z2
        @pl.when(i == 0)
        def _():
            s2_ref[...] = jnp.zeros_like(s2_ref)
            q2_ref[...] = jnp.zeros_like(q2_ref)
        s2_ref[...] += jnp.sum(z2, 0, keepdims=True)
        q2_ref[...] += jnp.sum(z2 * z2, 0, keepdims=True)

    return pl.pallas_call(
        body,
        grid=(nb,),
        in_specs=[
            pl.BlockSpec((BN_ROWS, h2), lambda i: (i, 0)),
            pl.BlockSpec((1, h2), lambda i: (0, 0)),
            pl.BlockSpec((1, h2), lambda i: (0, 0)),
            pl.BlockSpec((1, h2), lambda i: (0, 0)),
            pl.BlockSpec((1, h2), lambda i: (0, 0)),
            pl.BlockSpec((h2, ho), lambda i: (0, 0)),
            pl.BlockSpec((1, ho), lambda i: (0, 0)),
        ],
        out_specs=[
            pl.BlockSpec((BN_ROWS, ho), lambda i: (i, 0)),
            pl.BlockSpec((1, ho), lambda i: (0, 0)),
            pl.BlockSpec((1, ho), lambda i: (0, 0)),
        ],
        out_shape=[
            jax.ShapeDtypeStruct((n, ho), jnp.float32),
            jax.ShapeDtypeStruct((1, ho), jnp.float32),
            jax.ShapeDtypeStruct((1, ho), jnp.float32),
        ],
    )(z1, s1, q1, g1, bt1, w2, b2)


def _pass_c_split(z2, s2, q2, g, bt, *, n):
    h = z2.shape[1]
    ch = h // 2
    nb = n // BN_ROWS

    def body(z_ref, s_ref, q_ref, g_ref, bt_ref, out_ref):
        m = s_ref[...] / n
        v = q_ref[...] / n - m * m
        sc = g_ref[...] * lax.rsqrt(v + 1e-5)
        sh = bt_ref[...] - m * sc
        out_ref[...] = jnp.maximum(z_ref[...] * sc + sh, 0.0)

    return pl.pallas_call(
        body,
        grid=(nb, 2),
        in_specs=[
            pl.BlockSpec((BN_ROWS, ch), lambda i, hh: (i, hh)),
            pl.BlockSpec((1, ch), lambda i, hh: (0, hh)),
            pl.BlockSpec((1, ch), lambda i, hh: (0, hh)),
            pl.BlockSpec((1, ch), lambda i, hh: (0, hh)),
            pl.BlockSpec((1, ch), lambda i, hh: (0, hh)),
        ],
        out_specs=pl.BlockSpec((BN_ROWS, ch), lambda i, hh: (hh * nb + i, 0)),
        out_shape=jax.ShapeDtypeStruct((2 * n, ch), jnp.float32),
    )(z2, s2, q2, g, bt)


def _pass_c_pool(z2, s2, q2, g, bt, batch2d, *, n):
    h = z2.shape[1]
    nb = n // BN_ROWS

    def body(z_ref, s_ref, q_ref, g_ref, bt_ref, b_ref, out_ref):
        i = pl.program_id(0)
        m = s_ref[...] / n
        v = q_ref[...] / n - m * m
        sc = g_ref[...] * lax.rsqrt(v + 1e-5)
        sh = bt_ref[...] - m * sc
        hrows = jnp.maximum(z_ref[...] * sc + sh, 0.0)
        seg = lax.broadcasted_iota(jnp.int32, (G, BN_ROWS), 0)
        onehot = (seg == b_ref[0]).astype(jnp.float32)
        @pl.when(i == 0)
        def _():
            out_ref[...] = jnp.zeros_like(out_ref)
        out_ref[...] += jnp.dot(onehot, hrows, preferred_element_type=jnp.float32)

    return pl.pallas_call(
        body,
        grid=(nb,),
        in_specs=[
            pl.BlockSpec((BN_ROWS, h), lambda i: (i, 0)),
            pl.BlockSpec((1, h), lambda i: (0, 0)),
            pl.BlockSpec((1, h), lambda i: (0, 0)),
            pl.BlockSpec((1, h), lambda i: (0, 0)),
            pl.BlockSpec((1, h), lambda i: (0, 0)),
            pl.BlockSpec((1, 1, BN_ROWS), lambda i: (i, 0, 0)),
        ],
        out_specs=pl.BlockSpec((G, h), lambda i: (0, 0)),
        out_shape=jax.ShapeDtypeStruct((G, h), jnp.float32),
    )(z2, s2, q2, g, bt, batch2d)


def _head(pooled, w1, b1, g1, bt1, w2, b2):
    h = pooled.shape[1]
    out = w2.shape[1]

    def body(p_ref, w1_ref, b1_ref, g_ref, bt_ref, w2_ref, b2_ref, o_ref):
        y = jnp.dot(p_ref[...], w1_ref[...],
                    preferred_element_type=jnp.float32) + b1_ref[...]
        m = jnp.mean(y, 0, keepdims=True)
        v = jnp.mean(y * y, 0, keepdims=True) - m * m
        y = jnp.maximum(g_ref[...] * (y - m) * lax.rsqrt(v + 1e-5) + bt_ref[...], 0.0)
        y2 = jnp.dot(y, w2_ref[...], preferred_element_type=jnp.float32) + b2_ref[...]
        mx = jnp.max(y2, axis=-1, keepdims=True)
        lse = mx + jnp.log(jnp.sum(jnp.exp(y2 - mx), axis=-1, keepdims=True))
        o_ref[...] = y2 - lse

    return pl.pallas_call(
        body,
        out_shape=jax.ShapeDtypeStruct((G, out), jnp.float32),
    )(pooled, w1, b1, g1, bt1, w2, b2)


def kernel(x, edge_index, batch, params):
    n = x.shape[0]
    e = edge_index.shape[1]
    src = edge_index[0]
    dst = edge_index[1]
    grp = 4 * GN * K * NS * NC
    e2 = -(-e // grp) * grp
    src_p = jnp.concatenate([src, jnp.arange(e2 - e, dtype=jnp.int32) % n])
    dst_p = jnp.concatenate([dst, n + (jnp.arange(e2 - e, dtype=jnp.int32) % 16)])
    sb = src_p.reshape(-1, 1, K)
    db = dst_p.reshape(-1, 1, K)
    comb0 = jnp.concatenate([sb, db], 1)
    comb = jnp.concatenate([comb0,
                            jnp.concatenate([sb + n, db], 1)], 0)
    zeros = jnp.zeros((n // NS, 128), jnp.float32)
    batch3d = batch.reshape(n // BN_ROWS, 1, BN_ROWS)

    def row(a):
        return a.reshape(1, -1)

    hsplit = None
    z2 = s2 = q2 = None
    for i in range(3):
        p = params[f"conv{i}"]
        epsp = (1.0 + p["eps"]).reshape(1, 1)
        if i == 0:
            p01 = _sc_agg(x, comb0, zeros, n=n, edge_split=True)
            z1, s1, q1 = _pass_a0(x, p01, epsp, p["W1"], row(p["b1"]), n=n)
        else:
            hsplit = _pass_c_split(z2, s2, q2, row(params[f"bn{i-1}_g"]),
                                   row(params[f"bn{i-1}_b"]), n=n)
            agg = _sc_agg(hsplit, comb, zeros, n=n, edge_split=False)
            z1, s1, q1 = _pass_a(hsplit, agg, epsp, p["W1"], row(p["b1"]), n=n)
        z2, s2, q2 = _pass_b(z1, s1, q1, row(p["g1"]), row(p["bt1"]),
                             p["W2"], row(p["b2"]), n=n)

    pooled = _pass_c_pool(z2, s2, q2, row(params["bn2_g"]), row(params["bn2_b"]),
                          batch3d, n=n)
    return _head(pooled, params["lin1_W"], row(params["lin1_b"]),
                 row(params["bn1_g"]), row(params["bn1_b"]),
                 params["lin2_W"], row(params["lin2_b"]))


# speedup vs baseline: 1.0478x; 1.0466x over previous
import functools

import jax
import jax.numpy as jnp
from jax import lax
from jax.experimental import pallas as pl
from jax.experimental.pallas import tpu as pltpu
from jax.experimental.pallas import tpu_sc as plsc

G = 64
NS = 16
NC = 2
K = 128
BN_ROWS = 2000


def _sc_agg(table, comb, zeros, *, n, edge_split):
    ch = table.shape[1]
    ncht = comb.shape[0] if edge_split else comb.shape[0] // 2
    nch_w = ncht // (NS * NC) if edge_split else ncht // NS
    npair = nch_w // 2
    zr = (n // NS) & ~7
    tail = n - NS * zr
    mesh = plsc.VectorSubcoreMesh(core_axis_name="c", subcore_axis_name="s")

    @functools.partial(
        pl.kernel,
        out_type=jax.ShapeDtypeStruct((2 * n, ch), jnp.float32),
        mesh=mesh,
        scratch_types=[
            [pltpu.VMEM((2, K), jnp.int32)] * 2,
            [pltpu.VMEM((K, ch), jnp.float32)] * 2,
            pltpu.VMEM_SHARED((n + 16, ch), jnp.float32),
            [pltpu.SemaphoreType.DMA] * 2,
        ],
    )
    def body(comb_ref, t_ref, z_ref, out_ref, idx, rows, aggsh, sem):
        c = lax.axis_index("c")
        s = lax.axis_index("s")
        pltpu.sync_copy(z_ref.at[pl.ds(0, zr)],
                        aggsh.at[pl.ds(pl.multiple_of(s * zr, 8), zr)])
        @pl.when(s == 0)
        def _():
            pltpu.sync_copy(z_ref.at[pl.ds(0, tail)],
                            aggsh.at[pl.ds(NS * zr, tail)])
        plsc.subcore_barrier()

        if edge_split:
            base = (s * NC + c) * nch_w
        else:
            base = c * ncht + s * nch_w

        def load_idx(b, g):
            pltpu.sync_copy(comb_ref.at[g], idx[b])

        def fire_gather(b):
            pltpu.async_copy(t_ref.at[idx[b].at[0]], rows[b], sem[b])

        def wait_gather(b):
            pltpu.make_async_copy(t_ref.at[pl.ds(0, K)], rows[b], sem[b]).wait()

        def scat(b):
            pltpu.sync_copy(rows[b], aggsh.at[idx[b].at[1]], add=True)

        load_idx(0, base)
        fire_gather(0)

        def pair(jj, carry):
            g0 = base + 2 * jj
            load_idx(1, g0 + 1)
            fire_gather(1)
            wait_gather(0)
            scat(0)
            @pl.when(jj + 1 < nch_w // 2)
            def _():
                load_idx(0, g0 + 2)
                fire_gather(0)
            wait_gather(1)
            scat(1)
            return carry

        lax.fori_loop(0, nch_w // 2, pair, 0)
        plsc.subcore_barrier()
        pltpu.sync_copy(aggsh.at[pl.ds(pl.multiple_of(s * zr, 8), zr)],
                        out_ref.at[pl.ds(pl.multiple_of(c * n + s * zr, 8), zr)])
        @pl.when(s == 0)
        def _():
            pltpu.sync_copy(aggsh.at[pl.ds(NS * zr, tail)],
                            out_ref.at[pl.ds(pl.multiple_of(c * n + NS * zr, 8), tail)])

    return body(comb, table, zeros)


def _pass_a0(x, p01, epsp, w1, b1, *, n):
    cin = x.shape[1]
    h2 = w1.shape[1]
    nb = n // BN_ROWS

    def body(eps_ref, x_ref, a0_ref, a1_ref, w_ref, b_ref, z_ref, s_ref, q_ref):
        i = pl.program_id(0)
        u = x_ref[...] * eps_ref[0, 0] + a0_ref[...] + a1_ref[...]
        z = jnp.dot(u, w_ref[...], preferred_element_type=jnp.float32) + b_ref[...]
        z_ref[...] = z
        @pl.when(i == 0)
        def _():
            s_ref[...] = jnp.zeros_like(s_ref)
            q_ref[...] = jnp.zeros_like(q_ref)
        s_ref[...] += jnp.sum(z, 0, keepdims=True)
        q_ref[...] += jnp.sum(z * z, 0, keepdims=True)

    return pl.pallas_call(
        body,
        grid=(nb,),
        in_specs=[
            pl.BlockSpec((1, 1), lambda i: (0, 0), memory_space=pltpu.SMEM),
            pl.BlockSpec((BN_ROWS, cin), lambda i: (i, 0)),
            pl.BlockSpec((BN_ROWS, cin), lambda i: (i, 0)),
            pl.BlockSpec((BN_ROWS, cin), lambda i: (nb + i, 0)),
            pl.BlockSpec((cin, h2), lambda i: (0, 0)),
            pl.BlockSpec((1, h2), lambda i: (0, 0)),
        ],
        out_specs=[
            pl.BlockSpec((BN_ROWS, h2), lambda i: (i, 0)),
            pl.BlockSpec((1, h2), lambda i: (0, 0)),
            pl.BlockSpec((1, h2), lambda i: (0, 0)),
        ],
        out_shape=[
            jax.ShapeDtypeStruct((n, h2), jnp.float32),
            jax.ShapeDtypeStruct((1, h2), jnp.float32),
            jax.ShapeDtypeStruct((1, h2), jnp.float32),
        ],
    )(epsp, x, p01, p01, w1, b1)


def _pass_a(hsplit, agg, epsp, w1, b1, *, n):
    ch = hsplit.shape[1]
    h2 = w1.shape[1]
    nb = n // BN_ROWS

    def body(eps_ref, h0, h1, a0, a1, w_ref, b_ref, z_ref, s_ref, q_ref):
        i = pl.program_id(0)
        ep = eps_ref[0, 0]
        u0 = h0[...] * ep + a0[...]
        u1 = h1[...] * ep + a1[...]
        z = (jnp.dot(u0, w_ref[:ch], preferred_element_type=jnp.float32)
             + jnp.dot(u1, w_ref[ch:], preferred_element_type=jnp.float32)
             + b_ref[...])
        z_ref[...] = z
        @pl.when(i == 0)
        def _():
            s_ref[...] = jnp.zeros_like(s_ref)
            q_ref[...] = jnp.zeros_like(q_ref)
        s_ref[...] += jnp.sum(z, 0, keepdims=True)
        q_ref[...] += jnp.sum(z * z, 0, keepdims=True)

    return pl.pallas_call(
        body,
        grid=(nb,),
        in_specs=[
            pl.BlockSpec((1, 1), lambda i: (0, 0), memory_space=pltpu.SMEM),
            pl.BlockSpec((BN_ROWS, ch), lambda i: (i, 0)),
            pl.BlockSpec((BN_ROWS, ch), lambda i: (nb + i, 0)),
            pl.BlockSpec((BN_ROWS, ch), lambda i: (i, 0)),
            pl.BlockSpec((BN_ROWS, ch), lambda i: (nb + i, 0)),
            pl.BlockSpec((2 * ch, h2), lambda i: (0, 0)),
            pl.BlockSpec((1, h2), lambda i: (0, 0)),
        ],
        out_specs=[
            pl.BlockSpec((BN_ROWS, h2), lambda i: (i, 0)),
            pl.BlockSpec((1, h2), lambda i: (0, 0)),
            pl.BlockSpec((1, h2), lambda i: (0, 0)),
        ],
        out_shape=[
            jax.ShapeDtypeStruct((n, h2), jnp.float32),
            jax.ShapeDtypeStruct((1, h2), jnp.float32),
            jax.ShapeDtypeStruct((1, h2), jnp.float32),
        ],
    )(epsp, hsplit, hsplit, agg, agg, w1, b1)


def _pass_b(z1, s1, q1, g1, bt1, w2, b2, *, n):
    h2 = z1.shape[1]
    ho = w2.shape[1]
    nb = n // BN_ROWS

    def body(z1_ref, s_ref, q_ref, g_ref, bt_ref, w_ref, b_ref,
             z2_ref, s2_ref, q2_ref):
        i = pl.program_id(0)
        m = s_ref[...] / n
        v = q_ref[...] / n - m * m
        sc = g_ref[...] * lax.rsqrt(v + 1e-5)
        sh = bt_ref[...] - m * sc
        y = jnp.maximum(z1_ref[...] * sc + sh, 0.0)
        z2 = jnp.dot(y, w_ref[...], preferred_element_type=jnp.float32) + b_ref[...]
        z2_ref[...] = z2
        @pl.when(i == 0)
        def _():
            s2_ref[...] = jnp.zeros_like(s2_ref)
            q2_ref[...] = jnp.zeros_like(q2_ref)
        s2_ref[...] += jnp.sum(z2, 0, keepdims=True)
        q2_ref[...] += jnp.sum(z2 * z2, 0, keepdims=True)

    return pl.pallas_call(
        body,
        grid=(nb,),
        in_specs=[
            pl.BlockSpec((BN_ROWS, h2), lambda i: (i, 0)),
            pl.BlockSpec((1, h2), lambda i: (0, 0)),
            pl.BlockSpec((1, h2), lambda i: (0, 0)),
            pl.BlockSpec((1, h2), lambda i: (0, 0)),
            pl.BlockSpec((1, h2), lambda i: (0, 0)),
            pl.BlockSpec((h2, ho), lambda i: (0, 0)),
            pl.BlockSpec((1, ho), lambda i: (0, 0)),
        ],
        out_specs=[
            pl.BlockSpec((BN_ROWS, ho), lambda i: (i, 0)),
            pl.BlockSpec((1, ho), lambda i: (0, 0)),
            pl.BlockSpec((1, ho), lambda i: (0, 0)),
        ],
        out_shape=[
            jax.ShapeDtypeStruct((n, ho), jnp.float32),
            jax.ShapeDtypeStruct((1, ho), jnp.float32),
            jax.ShapeDtypeStruct((1, ho), jnp.float32),
        ],
    )(z1, s1, q1, g1, bt1, w2, b2)


def _pass_c_split(z2, s2, q2, g, bt, *, n):
    h = z2.shape[1]
    ch = h // 2
    nb = n // BN_ROWS

    def body(z_ref, s_ref, q_ref, g_ref, bt_ref, out_ref):
        m = s_ref[...] / n
        v = q_ref[...] / n - m * m
        sc = g_ref[...] * lax.rsqrt(v + 1e-5)
        sh = bt_ref[...] - m * sc
        out_ref[...] = jnp.maximum(z_ref[...] * sc + sh, 0.0)

    return pl.pallas_call(
        body,
        grid=(nb, 2),
        in_specs=[
            pl.BlockSpec((BN_ROWS, ch), lambda i, hh: (i, hh)),
            pl.BlockSpec((1, ch), lambda i, hh: (0, hh)),
            pl.BlockSpec((1, ch), lambda i, hh: (0, hh)),
            pl.BlockSpec((1, ch), lambda i, hh: (0, hh)),
            pl.BlockSpec((1, ch), lambda i, hh: (0, hh)),
        ],
        out_specs=pl.BlockSpec((BN_ROWS, ch), lambda i, hh: (hh * nb + i, 0)),
        out_shape=jax.ShapeDtypeStruct((2 * n, ch), jnp.float32),
    )(z2, s2, q2, g, bt)


def _pass_c_pool(z2, s2, q2, g, bt, batch2d, *, n):
    h = z2.shape[1]
    nb = n // BN_ROWS

    def body(z_ref, s_ref, q_ref, g_ref, bt_ref, b_ref, out_ref):
        i = pl.program_id(0)
        m = s_ref[...] / n
        v = q_ref[...] / n - m * m
        sc = g_ref[...] * lax.rsqrt(v + 1e-5)
        sh = bt_ref[...] - m * sc
        hrows = jnp.maximum(z_ref[...] * sc + sh, 0.0)
        seg = lax.broadcasted_iota(jnp.int32, (G, BN_ROWS), 0)
        onehot = (seg == b_ref[0]).astype(jnp.float32)
        @pl.when(i == 0)
        def _():
            out_ref[...] = jnp.zeros_like(out_ref)
        out_ref[...] += jnp.dot(onehot, hrows, preferred_element_type=jnp.float32)

    return pl.pallas_call(
        body,
        grid=(nb,),
        in_specs=[
            pl.BlockSpec((BN_ROWS, h), lambda i: (i, 0)),
            pl.BlockSpec((1, h), lambda i: (0, 0)),
            pl.BlockSpec((1, h), lambda i: (0, 0)),
            pl.BlockSpec((1, h), lambda i: (0, 0)),
            pl.BlockSpec((1, h), lambda i: (0, 0)),
            pl.BlockSpec((1, 1, BN_ROWS), lambda i: (i, 0, 0)),
        ],
        out_specs=pl.BlockSpec((G, h), lambda i: (0, 0)),
        out_shape=jax.ShapeDtypeStruct((G, h), jnp.float32),
    )(z2, s2, q2, g, bt, batch2d)


def _head(pooled, w1, b1, g1, bt1, w2, b2):
    h = pooled.shape[1]
    out = w2.shape[1]

    def body(p_ref, w1_ref, b1_ref, g_ref, bt_ref, w2_ref, b2_ref, o_ref):
        y = jnp.dot(p_ref[...], w1_ref[...],
                    preferred_element_type=jnp.float32) + b1_ref[...]
        m = jnp.mean(y, 0, keepdims=True)
        v = jnp.mean(y * y, 0, keepdims=True) - m * m
        y = jnp.maximum(g_ref[...] * (y - m) * lax.rsqrt(v + 1e-5) + bt_ref[...], 0.0)
        y2 = jnp.dot(y, w2_ref[...], preferred_element_type=jnp.float32) + b2_ref[...]
        mx = jnp.max(y2, axis=-1, keepdims=True)
        lse = mx + jnp.log(jnp.sum(jnp.exp(y2 - mx), axis=-1, keepdims=True))
        o_ref[...] = y2 - lse

    return pl.pallas_call(
        body,
        out_shape=jax.ShapeDtypeStruct((G, out), jnp.float32),
    )(pooled, w1, b1, g1, bt1, w2, b2)


def kernel(x, edge_index, batch, params):
    n = x.shape[0]
    e = edge_index.shape[1]
    src = edge_index[0]
    dst = edge_index[1]
    grp = 2 * K * NS * NC
    e2 = -(-e // grp) * grp
    src_p = jnp.concatenate([src, jnp.arange(e2 - e, dtype=jnp.int32) % n])
    dst_p = jnp.concatenate([dst, n + (jnp.arange(e2 - e, dtype=jnp.int32) % 16)])
    sb = src_p.reshape(-1, 1, K)
    db = dst_p.reshape(-1, 1, K)
    comb0 = jnp.concatenate([sb, db], 1)
    comb = jnp.concatenate([comb0,
                            jnp.concatenate([sb + n, db], 1)], 0)
    zeros = jnp.zeros((n // NS, 128), jnp.float32)
    batch3d = batch.reshape(n // BN_ROWS, 1, BN_ROWS)

    def row(a):
        return a.reshape(1, -1)

    hsplit = None
    z2 = s2 = q2 = None
    for i in range(3):
        p = params[f"conv{i}"]
        epsp = (1.0 + p["eps"]).reshape(1, 1)
        if i == 0:
            p01 = _sc_agg(x, comb0, zeros, n=n, edge_split=True)
            z1, s1, q1 = _pass_a0(x, p01, epsp, p["W1"], row(p["b1"]), n=n)
        else:
            hsplit = _pass_c_split(z2, s2, q2, row(params[f"bn{i-1}_g"]),
                                   row(params[f"bn{i-1}_b"]), n=n)
            agg = _sc_agg(hsplit, comb, zeros, n=n, edge_split=False)
            z1, s1, q1 = _pass_a(hsplit, agg, epsp, p["W1"], row(p["b1"]), n=n)
        z2, s2, q2 = _pass_b(z1, s1, q1, row(p["g1"]), row(p["bt1"]),
                             p["W2"], row(p["b2"]), n=n)

    pooled = _pass_c_pool(z2, s2, q2, row(params["bn2_g"]), row(params["bn2_b"]),
                          batch3d, n=n)
    return _head(pooled, params["lin1_W"], row(params["lin1_b"]),
                 row(params["bn1_g"]), row(params["bn1_b"]),
                 params["lin2_W"], row(params["lin2_b"]))
